# Initial kernel scaffold; baseline (speedup 1.0000x reference)
import jax
import jax.numpy as jnp
from jax.experimental import pallas as pl


def kernel(source, rng_seq, rng_idx):
    raise NotImplementedError("write your pallas kernel here")


# speedup vs baseline: 50.1615x; 50.1615x over previous
import functools

import jax
import jax.numpy as jnp
from jax import lax
from jax.experimental import pallas as pl
from jax.experimental.pallas import tpu as pltpu
from jax.experimental.pallas import tpu_sc as plsc

SRC_SHAPE = (16384, 1024)
N = SRC_SHAPE[0] * SRC_SHAPE[1]
NW = N // 4
NUM_WORKERS = 32
PER_WORKER = N // NUM_WORKERS
CHUNK = 16384
CHUNK_W = CHUNK // 4
NUM_CHUNKS = PER_WORKER // CHUNK

_mesh = plsc.VectorSubcoreMesh(core_axis_name="c", subcore_axis_name="s")


@functools.partial(
    pl.kernel,
    mesh=_mesh,
    compiler_params=pltpu.CompilerParams(needs_layout_passes=False),
    out_type=jax.ShapeDtypeStruct((NW,), jnp.int32),
    scratch_types=[
        pltpu.VMEM((256,), jnp.float32),
        pltpu.VMEM((CHUNK,), jnp.float32),
        pltpu.VMEM((CHUNK,), jnp.float32),
        pltpu.VMEM((CHUNK_W,), jnp.int32),
        pltpu.VMEM((CHUNK_W,), jnp.int32),
        pltpu.VMEM((CHUNK_W,), jnp.int32),
        pltpu.VMEM((CHUNK_W,), jnp.int32),
        pltpu.SemaphoreType.DMA,
        pltpu.SemaphoreType.DMA,
        pltpu.SemaphoreType.DMA,
        pltpu.SemaphoreType.DMA,
        pltpu.SemaphoreType.DMA,
        pltpu.SemaphoreType.DMA,
    ],
)
def _bsgen_sc(src_hbm, table_hbm, idx_hbm, out_hbm,
              table_v, src_v0, src_v1, idx_v0, idx_v1, out_v0, out_v1,
              sem_s0, sem_s1, sem_i0, sem_i1, sem_o0, sem_o1):
    wid = lax.axis_index("s") * 2 + lax.axis_index("c")
    base = wid * PER_WORKER
    base_w = wid * (PER_WORKER // 4)

    slots = (
        (src_v0, idx_v0, out_v0, sem_s0, sem_i0, sem_o0),
        (src_v1, idx_v1, out_v1, sem_s1, sem_i1, sem_o1),
    )

    def start_in(g, slot):
        src_v, idx_v, _, sem_s, sem_i, _ = slot
        pltpu.async_copy(src_hbm.at[pl.ds(base + g * CHUNK, CHUNK)], src_v, sem_s)
        pltpu.async_copy(idx_hbm.at[pl.ds(base_w + g * CHUNK_W, CHUNK_W)],
                         idx_v, sem_i)

    def wait_in(slot):
        src_v, idx_v, _, sem_s, sem_i, _ = slot
        pltpu.make_async_copy(src_hbm.at[pl.ds(base, CHUNK)], src_v, sem_s).wait()
        pltpu.make_async_copy(idx_hbm.at[pl.ds(base_w, CHUNK_W)], idx_v,
                              sem_i).wait()

    def start_out(g, slot):
        out_v, sem_o = slot[2], slot[5]
        pltpu.async_copy(out_v, out_hbm.at[pl.ds(base_w + g * CHUNK_W, CHUNK_W)],
                         sem_o)

    def wait_out(slot):
        out_v, sem_o = slot[2], slot[5]
        pltpu.make_async_copy(out_v, out_hbm.at[pl.ds(base_w, CHUNK_W)],
                              sem_o).wait()

    pltpu.sync_copy(table_hbm, table_v)

    iota4 = lax.iota(jnp.int32, 16) * 4

    def compute(slot):
        src_v, idx_v, out_v = slot[0], slot[1], slot[2]

        def inner(j, carry):
            woff = j * 16
            w = idx_v[pl.ds(woff, 16)]
            sbase = iota4 + j * 64
            acc = None
            for m in range(4):
                im = (jnp.right_shift(w, 8 * m) if m else w) & 255
                tm = plsc.load_gather(table_v, [im])
                sm = plsc.load_gather(src_v, [sbase + m] if m else [sbase])
                rm = jnp.where(sm > tm, jnp.int32(1 << (8 * m)), jnp.int32(0))
                acc = rm if acc is None else acc | rm
            out_v[pl.ds(woff, 16)] = acc
            return carry

        lax.fori_loop(0, CHUNK // 64, inner, 0)

    for b in range(2):
        start_in(b, slots[b])

    def pair_body(p, carry):
        for b in range(2):
            g = p * 2 + b
            slot = slots[b]
            wait_in(slot)

            @pl.when(g >= 2)
            def _():
                wait_out(slot)

            compute(slot)
            start_out(g, slot)

            @pl.when(g + 2 < NUM_CHUNKS)
            def _():
                start_in(g + 2, slot)
        return carry

    lax.fori_loop(0, NUM_CHUNKS // 2, pair_body, 0)

    for b in range(2):
        wait_out(slots[b])


def kernel(source, rng_seq, rng_idx):
    src = source.reshape(N)
    idx8 = rng_idx.reshape(NW, 4).astype(jnp.uint8)
    idx32 = lax.bitcast_convert_type(idx8, jnp.int32)
    out_w = _bsgen_sc(src, rng_seq, idx32)
    out8 = lax.bitcast_convert_type(out_w, jnp.uint8).astype(jnp.int8)
    return out8.reshape(SRC_SHAPE)


# speedup vs baseline: 592.8108x; 11.8180x over previous
import functools

import jax
import jax.numpy as jnp
from jax import lax
from jax.experimental import pallas as pl
from jax.experimental.pallas import tpu as pltpu
from jax.experimental.pallas import tpu_sc as plsc

SRC_SHAPE = (16384, 1024)
N = SRC_SHAPE[0] * SRC_SHAPE[1]
NUM_WORKERS = 32
PER_WORKER = N // NUM_WORKERS
CHUNK = 16384
NUM_CHUNKS = PER_WORKER // CHUNK

_mesh = plsc.VectorSubcoreMesh(core_axis_name="c", subcore_axis_name="s")


@functools.partial(
    pl.kernel,
    mesh=_mesh,
    compiler_params=pltpu.CompilerParams(needs_layout_passes=False),
    out_type=jax.ShapeDtypeStruct((N,), jnp.int8),
    scratch_types=[
        pltpu.VMEM((256,), jnp.float32),
        pltpu.VMEM((CHUNK,), jnp.float32),
        pltpu.VMEM((CHUNK,), jnp.float32),
        pltpu.VMEM((CHUNK,), jnp.int32),
        pltpu.VMEM((CHUNK,), jnp.int32),
        pltpu.VMEM((CHUNK,), jnp.int8),
        pltpu.VMEM((CHUNK,), jnp.int8),
        pltpu.SemaphoreType.DMA,
        pltpu.SemaphoreType.DMA,
        pltpu.SemaphoreType.DMA,
        pltpu.SemaphoreType.DMA,
        pltpu.SemaphoreType.DMA,
        pltpu.SemaphoreType.DMA,
    ],
)
def _bsgen_sc(src_hbm, table_hbm, idx_hbm, out_hbm,
              table_v, src_v0, src_v1, idx_v0, idx_v1, out_v0, out_v1,
              sem_s0, sem_s1, sem_i0, sem_i1, sem_o0, sem_o1):
    wid = lax.axis_index("s") * 2 + lax.axis_index("c")
    base = wid * PER_WORKER

    slots = (
        (src_v0, idx_v0, out_v0, sem_s0, sem_i0, sem_o0),
        (src_v1, idx_v1, out_v1, sem_s1, sem_i1, sem_o1),
    )

    def start_in(g, slot):
        src_v, idx_v, _, sem_s, sem_i, _ = slot
        off = base + g * CHUNK
        pltpu.async_copy(src_hbm.at[pl.ds(off, CHUNK)], src_v, sem_s)
        pltpu.async_copy(idx_hbm.at[pl.ds(off, CHUNK)], idx_v, sem_i)

    def wait_in(slot):
        src_v, idx_v, _, sem_s, sem_i, _ = slot
        pltpu.make_async_copy(src_hbm.at[pl.ds(base, CHUNK)], src_v, sem_s).wait()
        pltpu.make_async_copy(idx_hbm.at[pl.ds(base, CHUNK)], idx_v, sem_i).wait()

    def start_out(g, slot):
        out_v, sem_o = slot[2], slot[5]
        pltpu.async_copy(out_v, out_hbm.at[pl.ds(base + g * CHUNK, CHUNK)], sem_o)

    def wait_out(slot):
        out_v, sem_o = slot[2], slot[5]
        pltpu.make_async_copy(out_v, out_hbm.at[pl.ds(base, CHUNK)], sem_o).wait()

    pltpu.sync_copy(table_hbm, table_v)

    iota4 = lax.iota(jnp.int32, 16) * 4

    def compute(slot):
        src_v, idx_v, out_v = slot[0], slot[1], slot[2]

        def inner(j, carry):
            off = j * 64
            sbase = iota4 + off
            acc = None
            for m in range(4):
                pos = [sbase + m] if m else [sbase]
                im = plsc.load_gather(idx_v, pos)
                tm = plsc.load_gather(table_v, [im])
                sm = plsc.load_gather(src_v, pos)
                rm = jnp.where(sm > tm, jnp.int32(1 << (8 * m)), jnp.int32(0))
                acc = rm if acc is None else acc | rm
            out_v[pl.ds(off, 64)] = plsc.bitcast(acc, jnp.int8)
            return carry

        lax.fori_loop(0, CHUNK // 64, inner, 0)

    for b in range(2):
        start_in(b, slots[b])

    def pair_body(p, carry):
        for b in range(2):
            g = p * 2 + b
            slot = slots[b]
            wait_in(slot)

            @pl.when(g >= 2)
            def _():
                wait_out(slot)

            compute(slot)
            start_out(g, slot)

            @pl.when(g + 2 < NUM_CHUNKS)
            def _():
                start_in(g + 2, slot)
        return carry

    lax.fori_loop(0, NUM_CHUNKS // 2, pair_body, 0)

    for b in range(2):
        wait_out(slots[b])


def kernel(source, rng_seq, rng_idx):
    src = source.reshape(N)
    idx = rng_idx.reshape(N).astype(jnp.int32)
    out = _bsgen_sc(src, rng_seq, idx)
    return out.reshape(SRC_SHAPE)
